# Initial kernel scaffold; baseline (speedup 1.0000x reference)
import functools

import jax
import jax.numpy as jnp
from jax import lax
from jax.experimental import pallas as pl
from jax.experimental.pallas import tpu as pltpu
from jax.experimental.pallas import tpu_sc as plsc

N = 10000
E = 320000
D = 128

NC = 2
NS = 16
NW = NC * NS
EPW = E // NW
CH = 80
NCHUNK = EPW // CH
NPAD = 10240
DEG_PER_SUB = NPAD // NS
ROWS_PER_SUB = N // NS
ZROWS = 125

_mesh = plsc.VectorSubcoreMesh(core_axis_name="c", subcore_axis_name="s")


@functools.partial(
    pl.kernel,
    out_type=jax.ShapeDtypeStruct((NC, NPAD), jnp.float32),
    mesh=_mesh,
    scratch_types=[
        pltpu.VMEM((CH,), jnp.int32),
        pltpu.VMEM((CH,), jnp.float32),
        pltpu.VMEM((DEG_PER_SUB,), jnp.float32),
        pltpu.VMEM_SHARED((NPAD,), jnp.float32),
    ],
)
def _deg_kernel(dst_hbm, ew_hbm, out_hbm, dst_v, ew_v, zbuf_v, deg_sh):
    c = lax.axis_index("c")
    s = lax.axis_index("s")
    wid = s * NC + c

    def zb(i, carry):
        zbuf_v[pl.ds(i * 16, 16)] = jnp.zeros((16,), jnp.float32)
        return carry

    lax.fori_loop(0, DEG_PER_SUB // 16, zb, 0)
    pltpu.sync_copy(zbuf_v, deg_sh.at[pl.ds(s * DEG_PER_SUB, DEG_PER_SUB)])
    plsc.subcore_barrier()

    base = wid * EPW

    def chunk(ci, carry):
        off = base + ci * CH
        pltpu.sync_copy(dst_hbm.at[pl.ds(off, CH)], dst_v)
        pltpu.sync_copy(ew_hbm.at[pl.ds(off, CH)], ew_v)
        pltpu.sync_copy(ew_v, deg_sh.at[dst_v], add=True)
        return carry

    lax.fori_loop(0, NCHUNK, chunk, 0)
    plsc.subcore_barrier()
    pltpu.sync_copy(
        deg_sh.at[pl.ds(s * DEG_PER_SUB, DEG_PER_SUB)],
        out_hbm.at[c, pl.ds(s * DEG_PER_SUB, DEG_PER_SUB)],
    )


def _prop_body(compute_norm, src_hbm, dst_hbm, w_hbm, dis_hbm, feat_hbm,
               norm_out, agg_out, src_v, dst_v, w_v, norm_v, dis_v, rows_v,
               zb_v, agg_sh, sem):
    c = lax.axis_index("c")
    s = lax.axis_index("s")
    wid = s * NC + c

    def z(i, carry):
        zb_v[i // 8, pl.ds((i % 8) * 16, 16)] = jnp.zeros((16,), jnp.float32)
        return carry

    lax.fori_loop(0, ZROWS * 8, z, 0)
    for k in range(ROWS_PER_SUB // ZROWS):
        pltpu.sync_copy(zb_v, agg_sh.at[pl.ds(s * ROWS_PER_SUB + k * ZROWS, ZROWS)])
    if compute_norm:
        pltpu.sync_copy(dis_hbm, dis_v)
    plsc.subcore_barrier()

    base = wid * EPW

    def chunk(ci, carry):
        off = base + ci * CH
        pltpu.sync_copy(src_hbm.at[pl.ds(off, CH)], src_v)
        pltpu.sync_copy(dst_hbm.at[pl.ds(off, CH)], dst_v)
        pltpu.sync_copy(w_hbm.at[pl.ds(off, CH)], w_v)

        if compute_norm:
            def nrm(j, carry2):
                sl = pl.ds(j * 16, 16)
                sv = src_v[sl]
                dv = dst_v[sl]
                norm_v[sl] = (plsc.load_gather(dis_v, [sv]) * w_v[sl]
                              * plsc.load_gather(dis_v, [dv]))
                return carry2

            lax.fori_loop(0, CH // 16, nrm, 0)
            pltpu.sync_copy(norm_v, norm_out.at[pl.ds(off, CH)])
            nv = norm_v
        else:
            nv = w_v

        pltpu.async_copy(feat_hbm.at[src_v], rows_v, sem).wait()

        def scale(e, carry2):
            nsplat = plsc.load_gather(nv, [jnp.full((16,), e, jnp.int32)])
            for j in range(8):
                sl = pl.ds(j * 16, 16)
                rows_v[e, sl] = rows_v[e, sl] * nsplat
            return carry2

        lax.fori_loop(0, CH, scale, 0)
        pltpu.sync_copy(rows_v, agg_sh.at[dst_v], add=True)
        return carry

    lax.fori_loop(0, NCHUNK, chunk, 0)
    plsc.subcore_barrier()
    pltpu.sync_copy(
        agg_sh.at[pl.ds(s * ROWS_PER_SUB, ROWS_PER_SUB)],
        agg_out.at[c, pl.ds(s * ROWS_PER_SUB, ROWS_PER_SUB)],
    )


_prop1 = functools.partial(
    pl.kernel,
    out_type=(
        jax.ShapeDtypeStruct((E,), jnp.float32),
        jax.ShapeDtypeStruct((NC, N, D), jnp.float32),
    ),
    mesh=_mesh,
    scratch_types=[
        pltpu.VMEM((CH,), jnp.int32),
        pltpu.VMEM((CH,), jnp.int32),
        pltpu.VMEM((CH,), jnp.float32),
        pltpu.VMEM((CH,), jnp.float32),
        pltpu.VMEM((NPAD,), jnp.float32),
        pltpu.VMEM((CH, D), jnp.float32),
        pltpu.VMEM((ZROWS, D), jnp.float32),
        pltpu.VMEM_SHARED((N, D), jnp.float32),
        pltpu.SemaphoreType.DMA,
    ],
)(functools.partial(_prop_body, True))


def _prop2_body(src_hbm, dst_hbm, w_hbm, feat_hbm, agg_out, src_v, dst_v,
                w_v, norm_v, dis_v, rows_v, zb_v, agg_sh, sem):
    _prop_body(False, src_hbm, dst_hbm, w_hbm, None, feat_hbm, None,
               agg_out, src_v, dst_v, w_v, norm_v, dis_v, rows_v, zb_v,
               agg_sh, sem)


_prop2 = functools.partial(
    pl.kernel,
    out_type=jax.ShapeDtypeStruct((NC, N, D), jnp.float32),
    mesh=_mesh,
    scratch_types=[
        pltpu.VMEM((CH,), jnp.int32),
        pltpu.VMEM((CH,), jnp.int32),
        pltpu.VMEM((CH,), jnp.float32),
        pltpu.VMEM((CH,), jnp.float32),
        pltpu.VMEM((NPAD,), jnp.float32),
        pltpu.VMEM((CH, D), jnp.float32),
        pltpu.VMEM((ZROWS, D), jnp.float32),
        pltpu.VMEM_SHARED((N, D), jnp.float32),
        pltpu.SemaphoreType.DMA,
    ],
)(_prop2_body)


def _dis_body(d0_ref, d1_ref, o_ref):
    deg = d0_ref[...] + d1_ref[...]
    o_ref[...] = lax.rsqrt(jnp.where(deg > 0.0, deg, 1.0))


def _lin_relu_body(p0_ref, p1_ref, w_ref, b_ref, o_ref):
    agg = p0_ref[...] + p1_ref[...]
    y = jnp.dot(agg, w_ref[...], preferred_element_type=jnp.float32)
    o_ref[...] = jnp.maximum(y + b_ref[...], 0.0)


def _gru_body(p0_ref, p1_ref, x_ref, h_ref, w2_ref, b2_ref,
              wu0_ref, wu1_ref, wu2_ref, bu_ref,
              wr0_ref, wr1_ref, wr2_ref, br_ref,
              wc0_ref, wc1_ref, wc2_ref, bc_ref, o_ref):
    agg = p0_ref[...] + p1_ref[...]
    x = x_ref[...]
    h = h_ref[...]

    def mm(a, w_ref):
        return jnp.dot(a, w_ref[...], preferred_element_type=jnp.float32)

    g = jax.nn.sigmoid(mm(agg, w2_ref) + b2_ref[...])
    u = jax.nn.sigmoid(mm(x, wu0_ref) + mm(g, wu1_ref) + mm(h, wu2_ref)
                       + bu_ref[...])
    r = jax.nn.sigmoid(mm(x, wr0_ref) + mm(g, wr1_ref) + mm(h, wr2_ref)
                       + br_ref[...])
    cand = jnp.tanh(mm(x, wc0_ref) + mm(g, wc1_ref) + mm(r * h, wc2_ref)
                    + bc_ref[...])
    o_ref[...] = u * h + (1.0 - u) * cand


_ROWS_BLK = 1000


def _row_spec():
    return pl.BlockSpec((_ROWS_BLK, D), lambda i: (i, 0))


def _full_spec(shape):
    return pl.BlockSpec(shape, lambda i: tuple(0 for _ in shape))


@jax.jit
def kernel(x, edge_index, edge_weight, h,
           gcn_W1, gcn_b1, gcn_W2, gcn_b2,
           Wu, bu, Wr, br, Wc, bc):
    src = edge_index[0]
    dst = edge_index[1]

    deg_parts = _deg_kernel(dst, edge_weight)
    d0 = deg_parts[0].reshape(80, 128)
    d1 = deg_parts[1].reshape(80, 128)
    dis = pl.pallas_call(
        _dis_body,
        out_shape=jax.ShapeDtypeStruct((80, 128), jnp.float32),
    )(d0, d1).reshape(NPAD)

    norm, agg1_parts = _prop1(src, dst, edge_weight, dis, x)

    nblk = N // _ROWS_BLK
    h1 = pl.pallas_call(
        _lin_relu_body,
        grid=(nblk,),
        in_specs=[
            _row_spec(), _row_spec(),
            _full_spec((D, D)), _full_spec((1, D)),
        ],
        out_specs=_row_spec(),
        out_shape=jax.ShapeDtypeStruct((N, D), jnp.float32),
    )(agg1_parts[0], agg1_parts[1], gcn_W1, gcn_b1.reshape(1, D))

    agg2_parts = _prop2(src, dst, norm, h1)

    wu = [Wu[0:D], Wu[D:2 * D], Wu[2 * D:]]
    wr = [Wr[0:D], Wr[D:2 * D], Wr[2 * D:]]
    wc = [Wc[0:D], Wc[D:2 * D], Wc[2 * D:]]

    out = pl.pallas_call(
        _gru_body,
        grid=(nblk,),
        in_specs=[
            _row_spec(), _row_spec(), _row_spec(), _row_spec(),
            _full_spec((D, D)), _full_spec((1, D)),
            _full_spec((D, D)), _full_spec((D, D)), _full_spec((D, D)),
            _full_spec((1, D)),
            _full_spec((D, D)), _full_spec((D, D)), _full_spec((D, D)),
            _full_spec((1, D)),
            _full_spec((D, D)), _full_spec((D, D)), _full_spec((D, D)),
            _full_spec((1, D)),
        ],
        out_specs=_row_spec(),
        out_shape=jax.ShapeDtypeStruct((N, D), jnp.float32),
    )(agg2_parts[0], agg2_parts[1], x, h,
      gcn_W2, gcn_b2.reshape(1, D),
      wu[0], wu[1], wu[2], bu.reshape(1, D),
      wr[0], wr[1], wr[2], br.reshape(1, D),
      wc[0], wc[1], wc[2], bc.reshape(1, D))
    return out


# speedup vs baseline: 6.5808x; 6.5808x over previous
import functools

import jax
import jax.numpy as jnp
from jax import lax
from jax.experimental import pallas as pl
from jax.experimental.pallas import tpu as pltpu
from jax.experimental.pallas import tpu_sc as plsc

N = 10000
E = 320000
D = 128

NC = 2
NS = 16
NW = NC * NS
EPW = E // NW
CH = 80
NCHUNK = EPW // CH
NPAD = 10240
DEG_PER_SUB = NPAD // NS
NROW = 10240
AGG_PER_SUB = NROW // NS
ZROWS = 128

_mesh = plsc.VectorSubcoreMesh(core_axis_name="c", subcore_axis_name="s")


@functools.partial(
    pl.kernel,
    out_type=jax.ShapeDtypeStruct((NC * NPAD,), jnp.float32),
    mesh=_mesh,
    compiler_params=pltpu.CompilerParams(needs_layout_passes=False),
    scratch_types=[
        pltpu.VMEM((CH,), jnp.int32),
        pltpu.VMEM((CH,), jnp.float32),
        pltpu.VMEM((DEG_PER_SUB,), jnp.float32),
        pltpu.VMEM_SHARED((NPAD,), jnp.float32),
    ],
)
def _deg_kernel(dst_hbm, ew_hbm, out_hbm, dst_v, ew_v, zbuf_v, deg_sh):
    c = lax.axis_index("c")
    s = lax.axis_index("s")
    wid = s * NC + c

    def zb(i, carry):
        zbuf_v[pl.ds(i * 16, 16)] = jnp.zeros((16,), jnp.float32)
        return carry

    lax.fori_loop(0, DEG_PER_SUB // 16, zb, 0)
    pltpu.sync_copy(zbuf_v, deg_sh.at[pl.ds(s * DEG_PER_SUB, DEG_PER_SUB)])
    plsc.subcore_barrier()

    base = wid * EPW

    def chunk(ci, carry):
        off = base + ci * CH
        pltpu.sync_copy(dst_hbm.at[pl.ds(off, CH)], dst_v)
        pltpu.sync_copy(ew_hbm.at[pl.ds(off, CH)], ew_v)
        pltpu.sync_copy(ew_v, deg_sh.at[dst_v], add=True)
        return carry

    lax.fori_loop(0, NCHUNK, chunk, 0)
    plsc.subcore_barrier()

    @pl.when(s == 0)
    def _readout():
        pltpu.sync_copy(deg_sh, out_hbm.at[pl.ds(c * NPAD, NPAD)])


def _prop_body(compute_norm, src_hbm, dst_hbm, w_hbm, dis_hbm, feat_hbm,
               norm_out, agg_out, src_v, dst_v, w_v, norm_v, dis_v, rows_v,
               zb_v, agg_sh, sem):
    c = lax.axis_index("c")
    s = lax.axis_index("s")
    wid = s * NC + c

    def z(i, carry):
        zb_v[i // 8, pl.ds((i % 8) * 16, 16)] = jnp.zeros((16,), jnp.float32)
        return carry

    lax.fori_loop(0, ZROWS * 8, z, 0)
    for k in range(AGG_PER_SUB // ZROWS):
        pltpu.sync_copy(zb_v, agg_sh.at[pl.ds(s * AGG_PER_SUB + k * ZROWS, ZROWS)])
    if compute_norm:
        pltpu.sync_copy(dis_hbm, dis_v)
    plsc.subcore_barrier()

    base = wid * EPW

    def chunk(ci, carry):
        off = base + ci * CH
        pltpu.sync_copy(src_hbm.at[pl.ds(off, CH)], src_v)
        pltpu.sync_copy(dst_hbm.at[pl.ds(off, CH)], dst_v)
        pltpu.sync_copy(w_hbm.at[pl.ds(off, CH)], w_v)

        if compute_norm:
            def nrm(j, carry2):
                sl = pl.ds(j * 16, 16)
                sv = src_v[sl]
                dv = dst_v[sl]
                norm_v[sl] = (plsc.load_gather(dis_v, [sv]) * w_v[sl]
                              * plsc.load_gather(dis_v, [dv]))
                return carry2

            lax.fori_loop(0, CH // 16, nrm, 0)
            pltpu.sync_copy(norm_v, norm_out.at[pl.ds(off, CH)])
            nv = norm_v
        else:
            nv = w_v

        pltpu.async_copy(feat_hbm.at[src_v], rows_v, sem).wait()

        def scale(e, carry2):
            nsplat = plsc.load_gather(nv, [jnp.full((16,), e, jnp.int32)])
            for j in range(8):
                sl = pl.ds(j * 16, 16)
                rows_v[e, sl] = rows_v[e, sl] * nsplat
            return carry2

        lax.fori_loop(0, CH, scale, 0)
        pltpu.sync_copy(rows_v, agg_sh.at[dst_v], add=True)
        return carry

    lax.fori_loop(0, NCHUNK, chunk, 0)
    plsc.subcore_barrier()
    pltpu.sync_copy(
        agg_sh.at[pl.ds(s * AGG_PER_SUB, AGG_PER_SUB)],
        agg_out.at[c, pl.ds(s * AGG_PER_SUB, AGG_PER_SUB)],
    )


_prop1 = functools.partial(
    pl.kernel,
    out_type=(
        jax.ShapeDtypeStruct((E,), jnp.float32),
        jax.ShapeDtypeStruct((NC, NROW, D), jnp.float32),
    ),
    mesh=_mesh,
    compiler_params=pltpu.CompilerParams(needs_layout_passes=False),
    scratch_types=[
        pltpu.VMEM((CH,), jnp.int32),
        pltpu.VMEM((CH,), jnp.int32),
        pltpu.VMEM((CH,), jnp.float32),
        pltpu.VMEM((CH,), jnp.float32),
        pltpu.VMEM((NPAD,), jnp.float32),
        pltpu.VMEM((CH, D), jnp.float32),
        pltpu.VMEM((ZROWS, D), jnp.float32),
        pltpu.VMEM_SHARED((NROW, D), jnp.float32),
        pltpu.SemaphoreType.DMA,
    ],
)(functools.partial(_prop_body, True))


def _prop2_body(src_hbm, dst_hbm, w_hbm, feat_hbm, agg_out, src_v, dst_v,
                w_v, norm_v, dis_v, rows_v, zb_v, agg_sh, sem):
    _prop_body(False, src_hbm, dst_hbm, w_hbm, None, feat_hbm, None,
               agg_out, src_v, dst_v, w_v, norm_v, dis_v, rows_v, zb_v,
               agg_sh, sem)


_prop2 = functools.partial(
    pl.kernel,
    out_type=jax.ShapeDtypeStruct((NC, NROW, D), jnp.float32),
    mesh=_mesh,
    compiler_params=pltpu.CompilerParams(needs_layout_passes=False),
    scratch_types=[
        pltpu.VMEM((CH,), jnp.int32),
        pltpu.VMEM((CH,), jnp.int32),
        pltpu.VMEM((CH,), jnp.float32),
        pltpu.VMEM((CH,), jnp.float32),
        pltpu.VMEM((NPAD,), jnp.float32),
        pltpu.VMEM((CH, D), jnp.float32),
        pltpu.VMEM((ZROWS, D), jnp.float32),
        pltpu.VMEM_SHARED((NROW, D), jnp.float32),
        pltpu.SemaphoreType.DMA,
    ],
)(_prop2_body)


def _dis_body(d0_ref, d1_ref, o_ref):
    deg = d0_ref[...] + d1_ref[...]
    o_ref[...] = lax.rsqrt(jnp.where(deg > 0.0, deg, 1.0))


def _lin_relu_body(p0_ref, p1_ref, w_ref, b_ref, o_ref):
    agg = p0_ref[...] + p1_ref[...]
    y = jnp.dot(agg, w_ref[...], preferred_element_type=jnp.float32)
    o_ref[...] = jnp.maximum(y + b_ref[...], 0.0)


def _gru_body(p0_ref, p1_ref, x_ref, h_ref, w2_ref, b2_ref,
              wu0_ref, wu1_ref, wu2_ref, bu_ref,
              wr0_ref, wr1_ref, wr2_ref, br_ref,
              wc0_ref, wc1_ref, wc2_ref, bc_ref, o_ref):
    agg = p0_ref[...] + p1_ref[...]
    x = x_ref[...]
    h = h_ref[...]

    def mm(a, w_ref):
        return jnp.dot(a, w_ref[...], preferred_element_type=jnp.float32)

    g = jax.nn.sigmoid(mm(agg, w2_ref) + b2_ref[...])
    u = jax.nn.sigmoid(mm(x, wu0_ref) + mm(g, wu1_ref) + mm(h, wu2_ref)
                       + bu_ref[...])
    r = jax.nn.sigmoid(mm(x, wr0_ref) + mm(g, wr1_ref) + mm(h, wr2_ref)
                       + br_ref[...])
    cand = jnp.tanh(mm(x, wc0_ref) + mm(g, wc1_ref) + mm(r * h, wc2_ref)
                    + bc_ref[...])
    o_ref[...] = u * h + (1.0 - u) * cand


_ROWS_BLK = 1000


def _row_spec():
    return pl.BlockSpec((_ROWS_BLK, D), lambda i: (i, 0))


def _full_spec(shape):
    return pl.BlockSpec(shape, lambda i: tuple(0 for _ in shape))


@jax.jit
def kernel(x, edge_index, edge_weight, h,
           gcn_W1, gcn_b1, gcn_W2, gcn_b2,
           Wu, bu, Wr, br, Wc, bc):
    src = edge_index[0]
    dst = edge_index[1]

    deg_flat = _deg_kernel(dst, edge_weight)
    d0 = deg_flat[:NPAD].reshape(80, 128)
    d1 = deg_flat[NPAD:].reshape(80, 128)
    dis = pl.pallas_call(
        _dis_body,
        out_shape=jax.ShapeDtypeStruct((80, 128), jnp.float32),
    )(d0, d1).reshape(NPAD)

    norm, agg1_parts = _prop1(src, dst, edge_weight, dis, x)

    nblk = N // _ROWS_BLK
    h1 = pl.pallas_call(
        _lin_relu_body,
        grid=(nblk,),
        in_specs=[
            _row_spec(), _row_spec(),
            _full_spec((D, D)), _full_spec((1, D)),
        ],
        out_specs=_row_spec(),
        out_shape=jax.ShapeDtypeStruct((N, D), jnp.float32),
    )(agg1_parts[0], agg1_parts[1], gcn_W1, gcn_b1.reshape(1, D))

    agg2_parts = _prop2(src, dst, norm, h1)

    wu = [Wu[0:D], Wu[D:2 * D], Wu[2 * D:]]
    wr = [Wr[0:D], Wr[D:2 * D], Wr[2 * D:]]
    wc = [Wc[0:D], Wc[D:2 * D], Wc[2 * D:]]

    out = pl.pallas_call(
        _gru_body,
        grid=(nblk,),
        in_specs=[
            _row_spec(), _row_spec(), _row_spec(), _row_spec(),
            _full_spec((D, D)), _full_spec((1, D)),
            _full_spec((D, D)), _full_spec((D, D)), _full_spec((D, D)),
            _full_spec((1, D)),
            _full_spec((D, D)), _full_spec((D, D)), _full_spec((D, D)),
            _full_spec((1, D)),
            _full_spec((D, D)), _full_spec((D, D)), _full_spec((D, D)),
            _full_spec((1, D)),
        ],
        out_specs=_row_spec(),
        out_shape=jax.ShapeDtypeStruct((N, D), jnp.float32),
    )(agg2_parts[0], agg2_parts[1], x, h,
      gcn_W2, gcn_b2.reshape(1, D),
      wu[0], wu[1], wu[2], bu.reshape(1, D),
      wr[0], wr[1], wr[2], br.reshape(1, D),
      wc[0], wc[1], wc[2], bc.reshape(1, D))
    return out


# speedup vs baseline: 15.9049x; 2.4169x over previous
import functools

import jax
import jax.numpy as jnp
from jax import lax
from jax.experimental import pallas as pl
from jax.experimental.pallas import tpu as pltpu
from jax.experimental.pallas import tpu_sc as plsc

N = 10000
E = 320000
D = 128

NC = 2
NS = 16
NW = NC * NS
EPW = E // NW
CH = 80
NCHUNK = 128
EPW_PAD = NCHUNK * CH
GCH = 8
NGRP = NCHUNK // GCH
NBUF = 4
NPAD = 10240
DEG_PER_SUB = NPAD // NS
NROW = 10240
AGG_PER_SUB = NROW // NS

_mesh = plsc.VectorSubcoreMesh(core_axis_name="c", subcore_axis_name="s")


DCH = 80
DNCHUNK = EPW // DCH


@functools.partial(
    pl.kernel,
    out_type=jax.ShapeDtypeStruct((NC * NPAD,), jnp.float32),
    mesh=_mesh,
    compiler_params=pltpu.CompilerParams(needs_layout_passes=False),
    scratch_types=[
        pltpu.VMEM((DCH,), jnp.int32),
        pltpu.VMEM((DCH,), jnp.float32),
        pltpu.VMEM((DEG_PER_SUB,), jnp.float32),
        pltpu.VMEM_SHARED((NPAD,), jnp.float32),
    ],
)
def _deg_kernel(dst_hbm, ew_hbm, out_hbm, dst_v, ew_v, zbuf_v, deg_sh):
    c = lax.axis_index("c")
    s = lax.axis_index("s")
    wid = s * NC + c

    def zb(i, carry):
        zbuf_v[pl.ds(i * 16, 16)] = jnp.zeros((16,), jnp.float32)
        return carry

    lax.fori_loop(0, DEG_PER_SUB // 16, zb, 0)
    pltpu.sync_copy(zbuf_v, deg_sh.at[pl.ds(s * DEG_PER_SUB, DEG_PER_SUB)])
    plsc.subcore_barrier()

    base = wid * EPW

    def chunk(ci, carry):
        off = base + ci * DCH
        pltpu.sync_copy(dst_hbm.at[pl.ds(off, DCH)], dst_v)
        pltpu.sync_copy(ew_hbm.at[pl.ds(off, DCH)], ew_v)
        pltpu.sync_copy(ew_v, deg_sh.at[dst_v], add=True)
        return carry

    lax.fori_loop(0, DNCHUNK, chunk, 0)
    plsc.subcore_barrier()

    @pl.when(s == 0)
    def _readout():
        pltpu.sync_copy(deg_sh, out_hbm.at[pl.ds(c * NPAD, NPAD)])


def _prop_body(epk, feat_hbm, znd_hbm, agg_out,
               eg0, eg1, r0, r1, r2, r3, x0, x1, x2, x3,
               g0, g1, g2, g3, s0, s1, s2, s3, e0, e1, agg_sh):
    c = lax.axis_index("c")
    s = lax.axis_index("s")
    wid = s * NC + c
    egs = [eg0, eg1]
    rows = [r0, r1, r2, r3]
    sidx = [x0, x1, x2, x3]
    gsem = [g0, g1, g2, g3]
    ssem = [s0, s1, s2, s3]
    esem = [e0, e1]

    sl_sub = pl.ds(s * AGG_PER_SUB, AGG_PER_SUB)
    pltpu.sync_copy(znd_hbm.at[sl_sub], agg_sh.at[sl_sub])

    def e_start(p, g):
        pltpu.async_copy(epk.at[wid, g], egs[p], esem[p])

    def e_wait(p):
        pltpu.make_async_copy(epk.at[wid, 0], egs[p], esem[p]).wait()

    def g_start(b, p, row):
        pltpu.async_copy(feat_hbm.at[egs[p].at[row]], rows[b], gsem[b])

    def g_wait(b):
        pltpu.make_async_copy(feat_hbm.at[eg0.at[0]], rows[b], gsem[b]).wait()

    def s_start(b):
        pltpu.async_copy(rows[b], agg_sh.at[sidx[b]], ssem[b], add=True)

    def s_wait(b):
        pltpu.make_async_copy(rows[b], agg_sh.at[sidx[b]], ssem[b]).wait()

    e_start(0, 0)
    e_wait(0)
    plsc.subcore_barrier()
    g_start(0, 0, 0)
    g_start(1, 0, 3)

    def super_body(t, carry):
        for half in range(2):
            p = half
            for j in range(GCH):
                cc = t * 16 + half * 8 + j
                b = (half * 8 + j) % NBUF
                b2 = (b + 2) % NBUF

                if j == 0:
                    if half == 0:
                        e_start(1, 2 * t + 1)
                    else:
                        @pl.when(t < 7)
                        def _en():
                            e_start(0, 2 * t + 2)
                if j == 6:
                    if half == 0:
                        e_wait(1)
                    else:
                        @pl.when(t < 7)
                        def _ew():
                            e_wait(0)

                pc_j = j + 2
                if pc_j < GCH:
                    prow = 3 * pc_j
                    pparity = p
                else:
                    prow = 3 * (pc_j - GCH)
                    pparity = (p + 1) % 2

                @pl.when(cc + 2 < NCHUNK)
                def _prefetch():
                    @pl.when(cc >= 2)
                    def _drain():
                        s_wait(b2)
                    g_start(b2, pparity, prow)

                g_wait(b)

                drow = 3 * j + 1
                wrow = 3 * j + 2
                for q in range(CH // 16):
                    sl = pl.ds(q * 16, 16)
                    sidx[b][sl] = egs[p][drow, sl]

                def scale(e_, carry2):
                    wsplat = plsc.bitcast(
                        plsc.load_gather(
                            egs[p],
                            [jnp.full((16,), wrow, jnp.int32),
                             jnp.full((16,), e_, jnp.int32)]),
                        jnp.float32)
                    for q in range(8):
                        sl = pl.ds(q * 16, 16)
                        rows[b][e_, sl] = rows[b][e_, sl] * wsplat
                    return carry2

                lax.fori_loop(0, CH, scale, 0, unroll=2)
                s_start(b)
        return carry

    lax.fori_loop(0, NCHUNK // 16, super_body, 0)
    for b in range(NBUF):
        s_wait(b)
    plsc.subcore_barrier()
    pltpu.sync_copy(agg_sh.at[sl_sub], agg_out.at[c, sl_sub])


_prop = functools.partial(
    pl.kernel,
    out_type=jax.ShapeDtypeStruct((NC, NROW, D), jnp.float32),
    mesh=_mesh,
    compiler_params=pltpu.CompilerParams(needs_layout_passes=False),
    scratch_types=[pltpu.VMEM((3 * GCH, CH), jnp.int32)] * 2
    + [pltpu.VMEM((CH, D), jnp.float32)] * NBUF
    + [pltpu.VMEM((CH,), jnp.int32)] * NBUF
    + [pltpu.SemaphoreType.DMA] * (2 * NBUF + 2)
    + [pltpu.VMEM_SHARED((NROW, D), jnp.float32)],
)(_prop_body)


def _dis_body(d0_ref, d1_ref, o_ref):
    deg = d0_ref[...] + d1_ref[...]
    o_ref[...] = lax.rsqrt(jnp.where(deg > 0.0, deg, 1.0))


def _xscale_body(x_ref, disc_ref, o_ref):
    o_ref[...] = x_ref[...] * disc_ref[...]


def _lin_relu_body(p0_ref, p1_ref, disc_ref, w_ref, b_ref, o_ref):
    disc = disc_ref[...]
    agg = (p0_ref[...] + p1_ref[...]) * disc
    y = jnp.dot(agg, w_ref[...], preferred_element_type=jnp.float32)
    o_ref[...] = jnp.maximum(y + b_ref[...], 0.0) * disc


def _gru_body(p0_ref, p1_ref, disc_ref, x_ref, h_ref, w2_ref, b2_ref,
              wu0_ref, wu1_ref, wu2_ref, bu_ref,
              wr0_ref, wr1_ref, wr2_ref, br_ref,
              wc0_ref, wc1_ref, wc2_ref, bc_ref, o_ref):
    agg = (p0_ref[...] + p1_ref[...]) * disc_ref[...]
    x = x_ref[...]
    h = h_ref[...]

    def mm(a, w_ref):
        return jnp.dot(a, w_ref[...], preferred_element_type=jnp.float32)

    g = jax.nn.sigmoid(mm(agg, w2_ref) + b2_ref[...])
    u = jax.nn.sigmoid(mm(x, wu0_ref) + mm(g, wu1_ref) + mm(h, wu2_ref)
                       + bu_ref[...])
    r = jax.nn.sigmoid(mm(x, wr0_ref) + mm(g, wr1_ref) + mm(h, wr2_ref)
                       + br_ref[...])
    cand = jnp.tanh(mm(x, wc0_ref) + mm(g, wc1_ref) + mm(r * h, wc2_ref)
                    + bc_ref[...])
    o_ref[...] = u * h + (1.0 - u) * cand


_ROWS_BLK = 1000


def _row_spec():
    return pl.BlockSpec((_ROWS_BLK, D), lambda i: (i, 0))


def _col_spec():
    return pl.BlockSpec((_ROWS_BLK, 1), lambda i: (i, 0))


def _full_spec(shape):
    return pl.BlockSpec(shape, lambda i: tuple(0 for _ in shape))


@jax.jit
def kernel(x, edge_index, edge_weight, h,
           gcn_W1, gcn_b1, gcn_W2, gcn_b2,
           Wu, bu, Wr, br, Wc, bc):
    src = edge_index[0]
    dst = edge_index[1]

    deg_flat = _deg_kernel(dst, edge_weight)
    d0 = deg_flat[:NPAD].reshape(80, 128)
    d1 = deg_flat[NPAD:].reshape(80, 128)
    dis = pl.pallas_call(
        _dis_body,
        out_shape=jax.ShapeDtypeStruct((80, 128), jnp.float32),
    )(d0, d1)
    disc = dis.reshape(NROW, 1)

    pad_w = EPW_PAD - EPW
    fill_s = jnp.broadcast_to((jnp.arange(pad_w, dtype=jnp.int32) * 37) % N,
                              (NW, pad_w))
    fill_d = jnp.broadcast_to((jnp.arange(pad_w, dtype=jnp.int32) * 53) % NROW,
                              (NW, pad_w))
    src2 = jnp.concatenate([src.reshape(NW, EPW), fill_s], axis=1)
    dst2 = jnp.concatenate([dst.reshape(NW, EPW), fill_d], axis=1)
    w2 = jnp.concatenate(
        [edge_weight.reshape(NW, EPW), jnp.zeros((NW, pad_w), jnp.float32)],
        axis=1)
    wbits = lax.bitcast_convert_type(w2, jnp.int32)
    epk = jnp.stack(
        [src2.reshape(NW, NGRP, GCH, CH),
         dst2.reshape(NW, NGRP, GCH, CH),
         wbits.reshape(NW, NGRP, GCH, CH)], axis=3)
    epk = epk.reshape(NW, NGRP, 3 * GCH, CH)

    znd = jnp.zeros((NROW, D), jnp.float32)

    nblk = N // _ROWS_BLK
    xp = pl.pallas_call(
        _xscale_body,
        grid=(nblk,),
        in_specs=[_row_spec(), _col_spec()],
        out_specs=_row_spec(),
        out_shape=jax.ShapeDtypeStruct((N, D), jnp.float32),
    )(x, disc)

    agg1_parts = _prop(epk, xp, znd)

    h1p = pl.pallas_call(
        _lin_relu_body,
        grid=(nblk,),
        in_specs=[
            _row_spec(), _row_spec(), _col_spec(),
            _full_spec((D, D)), _full_spec((1, D)),
        ],
        out_specs=_row_spec(),
        out_shape=jax.ShapeDtypeStruct((N, D), jnp.float32),
    )(agg1_parts[0], agg1_parts[1], disc, gcn_W1, gcn_b1.reshape(1, D))

    agg2_parts = _prop(epk, h1p, znd)

    wu = [Wu[0:D], Wu[D:2 * D], Wu[2 * D:]]
    wr = [Wr[0:D], Wr[D:2 * D], Wr[2 * D:]]
    wc = [Wc[0:D], Wc[D:2 * D], Wc[2 * D:]]

    out = pl.pallas_call(
        _gru_body,
        grid=(nblk,),
        in_specs=[
            _row_spec(), _row_spec(), _col_spec(), _row_spec(), _row_spec(),
            _full_spec((D, D)), _full_spec((1, D)),
            _full_spec((D, D)), _full_spec((D, D)), _full_spec((D, D)),
            _full_spec((1, D)),
            _full_spec((D, D)), _full_spec((D, D)), _full_spec((D, D)),
            _full_spec((1, D)),
            _full_spec((D, D)), _full_spec((D, D)), _full_spec((D, D)),
            _full_spec((1, D)),
        ],
        out_specs=_row_spec(),
        out_shape=jax.ShapeDtypeStruct((N, D), jnp.float32),
    )(agg2_parts[0], agg2_parts[1], disc, x, h,
      gcn_W2, gcn_b2.reshape(1, D),
      wu[0], wu[1], wu[2], bu.reshape(1, D),
      wr[0], wr[1], wr[2], br.reshape(1, D),
      wc[0], wc[1], wc[2], bc.reshape(1, D))
    return out


# speedup vs baseline: 19.6993x; 1.2386x over previous
import functools

import jax
import jax.numpy as jnp
from jax import lax
from jax.experimental import pallas as pl
from jax.experimental.pallas import tpu as pltpu
from jax.experimental.pallas import tpu_sc as plsc

N = 10000
E = 320000
D = 128

NC = 2
NS = 16
NW = NC * NS
EPW = E // NW
CH = 80
NCHUNK = 128
EPW_PAD = NCHUNK * CH
GCH = 8
NGRP = NCHUNK // GCH
NBUF = 4
NPAD = 10240
DEG_PER_SUB = NPAD // NS
NROW = 10240
AGG_PER_SUB = NROW // NS

_mesh = plsc.VectorSubcoreMesh(core_axis_name="c", subcore_axis_name="s")


DROWS = NPAD * 8 // 128


@functools.partial(
    pl.kernel,
    out_type=jax.ShapeDtypeStruct((NW, DROWS, 128), jnp.float32),
    mesh=_mesh,
    compiler_params=pltpu.CompilerParams(needs_layout_passes=False),
    scratch_types=[
        pltpu.VMEM((80, 128), jnp.int32),
        pltpu.VMEM((80, 128), jnp.float32),
        pltpu.VMEM((DROWS, 128), jnp.float32),
    ],
)
def _deg_kernel(dst_hbm, ew_hbm, out_hbm, dst_v, ew_v, deg8):
    c = lax.axis_index("c")
    s = lax.axis_index("s")
    wid = s * NC + c
    pltpu.sync_copy(dst_hbm.at[wid], dst_v)
    pltpu.sync_copy(ew_hbm.at[wid], ew_v)

    zero16 = jnp.zeros((16,), jnp.float32)

    def z(i, carry):
        for q in range(8):
            deg8[i, pl.ds(q * 16, 16)] = zero16
        return carry

    lax.fori_loop(0, DROWS, z, 0)

    iota = lax.iota(jnp.int32, 16)
    col = iota & 7
    mlow = iota < 8
    mhigh = jnp.logical_not(mlow)

    def acc(i, carry):
        r = i // 8
        q = i % 8
        sl = pl.ds(q * 16, 16)
        dv = dst_v[r, sl]
        wv = ew_v[r, sl]
        addr = dv * 8 + col
        i0 = lax.shift_right_logical(addr, 7)
        i1 = addr & 127
        plsc.addupdate_scatter(deg8, [i0, i1], wv, mask=mlow)
        plsc.addupdate_scatter(deg8, [i0, i1], wv, mask=mhigh)
        return carry

    lax.fori_loop(0, 640, acc, 0, unroll=2)
    pltpu.sync_copy(deg8, out_hbm.at[wid])


def _prop_body(epk, feat_hbm, znd_hbm, agg_out,
               eg0, eg1, r0, r1, r2, r3, x0, x1, x2, x3,
               g0, g1, g2, g3, s0, s1, s2, s3, e0, e1, agg_sh):
    c = lax.axis_index("c")
    s = lax.axis_index("s")
    wid = s * NC + c
    egs = [eg0, eg1]
    rows = [r0, r1, r2, r3]
    sidx = [x0, x1, x2, x3]
    gsem = [g0, g1, g2, g3]
    ssem = [s0, s1, s2, s3]
    esem = [e0, e1]

    sl_sub = pl.ds(s * AGG_PER_SUB, AGG_PER_SUB)
    pltpu.sync_copy(znd_hbm.at[sl_sub], agg_sh.at[sl_sub])

    def e_start(p, g):
        pltpu.async_copy(epk.at[wid, g], egs[p], esem[p])

    def e_wait(p):
        pltpu.make_async_copy(epk.at[wid, 0], egs[p], esem[p]).wait()

    def g_start(b, p, row):
        pltpu.async_copy(feat_hbm.at[egs[p].at[row]], rows[b], gsem[b])

    def g_wait(b):
        pltpu.make_async_copy(feat_hbm.at[eg0.at[0]], rows[b], gsem[b]).wait()

    def s_start(b):
        pltpu.async_copy(rows[b], agg_sh.at[sidx[b]], ssem[b], add=True)

    def s_wait(b):
        pltpu.make_async_copy(rows[b], agg_sh.at[sidx[b]], ssem[b]).wait()

    e_start(0, 0)
    e_wait(0)
    plsc.subcore_barrier()
    g_start(0, 0, 0)
    g_start(1, 0, 3)

    def super_body(t, carry):
        for half in range(2):
            p = half
            for j in range(GCH):
                cc = t * 16 + half * 8 + j
                b = (half * 8 + j) % NBUF
                b2 = (b + 2) % NBUF

                if j == 0:
                    if half == 0:
                        e_start(1, 2 * t + 1)
                    else:
                        @pl.when(t < 7)
                        def _en():
                            e_start(0, 2 * t + 2)
                if j == 6:
                    if half == 0:
                        e_wait(1)
                    else:
                        @pl.when(t < 7)
                        def _ew():
                            e_wait(0)

                pc_j = j + 2
                if pc_j < GCH:
                    prow = 3 * pc_j
                    pparity = p
                else:
                    prow = 3 * (pc_j - GCH)
                    pparity = (p + 1) % 2

                @pl.when(cc + 2 < NCHUNK)
                def _prefetch():
                    @pl.when(cc >= 2)
                    def _drain():
                        s_wait(b2)
                    g_start(b2, pparity, prow)

                g_wait(b)

                drow = 3 * j + 1
                wrow = 3 * j + 2
                for q in range(CH // 16):
                    sl = pl.ds(q * 16, 16)
                    sidx[b][sl] = egs[p][drow, sl]

                def scale(e_, carry2):
                    wsplat = plsc.bitcast(
                        plsc.load_gather(
                            egs[p],
                            [jnp.full((16,), wrow, jnp.int32),
                             jnp.full((16,), e_, jnp.int32)]),
                        jnp.float32)
                    for q in range(8):
                        sl = pl.ds(q * 16, 16)
                        rows[b][e_, sl] = rows[b][e_, sl] * wsplat
                    return carry2

                lax.fori_loop(0, CH, scale, 0, unroll=2)
                s_start(b)
        return carry

    lax.fori_loop(0, NCHUNK // 16, super_body, 0)
    for b in range(NBUF):
        s_wait(b)
    plsc.subcore_barrier()
    pltpu.sync_copy(agg_sh.at[sl_sub], agg_out.at[c, sl_sub])


_prop = functools.partial(
    pl.kernel,
    out_type=jax.ShapeDtypeStruct((NC, NROW, D), jnp.float32),
    mesh=_mesh,
    compiler_params=pltpu.CompilerParams(needs_layout_passes=False),
    scratch_types=[pltpu.VMEM((3 * GCH, CH), jnp.int32)] * 2
    + [pltpu.VMEM((CH, D), jnp.float32)] * NBUF
    + [pltpu.VMEM((CH,), jnp.int32)] * NBUF
    + [pltpu.SemaphoreType.DMA] * (2 * NBUF + 2)
    + [pltpu.VMEM_SHARED((NROW, D), jnp.float32)],
)(_prop_body)


def _degred_body(degp_ref, s_ref, o_ref):
    acc = jnp.sum(degp_ref[...], axis=0)
    deg = jnp.dot(acc, s_ref[...], preferred_element_type=jnp.float32)
    o_ref[...] = lax.rsqrt(jnp.where(deg > 0.0, deg, 1.0))


def _xscale_body(x_ref, disc_ref, o_ref):
    o_ref[...] = x_ref[...] * disc_ref[...]


def _lin_relu_body(p0_ref, p1_ref, disc_ref, w_ref, b_ref, o_ref):
    disc = disc_ref[...]
    agg = (p0_ref[...] + p1_ref[...]) * disc
    y = jnp.dot(agg, w_ref[...], preferred_element_type=jnp.float32)
    o_ref[...] = jnp.maximum(y + b_ref[...], 0.0) * disc


def _gru_body(p0_ref, p1_ref, disc_ref, x_ref, h_ref, w2_ref, b2_ref,
              wu0_ref, wu1_ref, wu2_ref, bu_ref,
              wr0_ref, wr1_ref, wr2_ref, br_ref,
              wc0_ref, wc1_ref, wc2_ref, bc_ref, o_ref):
    agg = (p0_ref[...] + p1_ref[...]) * disc_ref[...]
    x = x_ref[...]
    h = h_ref[...]

    def mm(a, w_ref):
        return jnp.dot(a, w_ref[...], preferred_element_type=jnp.float32)

    g = jax.nn.sigmoid(mm(agg, w2_ref) + b2_ref[...])
    u = jax.nn.sigmoid(mm(x, wu0_ref) + mm(g, wu1_ref) + mm(h, wu2_ref)
                       + bu_ref[...])
    r = jax.nn.sigmoid(mm(x, wr0_ref) + mm(g, wr1_ref) + mm(h, wr2_ref)
                       + br_ref[...])
    cand = jnp.tanh(mm(x, wc0_ref) + mm(g, wc1_ref) + mm(r * h, wc2_ref)
                    + bc_ref[...])
    o_ref[...] = u * h + (1.0 - u) * cand


_ROWS_BLK = 1000


def _row_spec():
    return pl.BlockSpec((_ROWS_BLK, D), lambda i: (i, 0))


def _col_spec():
    return pl.BlockSpec((_ROWS_BLK, 1), lambda i: (i, 0))


def _full_spec(shape):
    return pl.BlockSpec(shape, lambda i: tuple(0 for _ in shape))


@jax.jit
def kernel(x, edge_index, edge_weight, h,
           gcn_W1, gcn_b1, gcn_W2, gcn_b2,
           Wu, bu, Wr, br, Wc, bc):
    src = edge_index[0]
    dst = edge_index[1]

    pad_w = EPW_PAD - EPW
    fill_s = jnp.broadcast_to((jnp.arange(pad_w, dtype=jnp.int32) * 37) % N,
                              (NW, pad_w))
    fill_d = jnp.broadcast_to((jnp.arange(pad_w, dtype=jnp.int32) * 53) % NROW,
                              (NW, pad_w))
    src2 = jnp.concatenate([src.reshape(NW, EPW), fill_s], axis=1)
    dst2 = jnp.concatenate([dst.reshape(NW, EPW), fill_d], axis=1)
    w2 = jnp.concatenate(
        [edge_weight.reshape(NW, EPW), jnp.zeros((NW, pad_w), jnp.float32)],
        axis=1)
    wbits = lax.bitcast_convert_type(w2, jnp.int32)

    degp = _deg_kernel(dst2.reshape(NW, 80, 128), w2.reshape(NW, 80, 128))
    smat = (jnp.arange(128, dtype=jnp.int32)[:, None] // 8
            == jnp.arange(16, dtype=jnp.int32)[None, :]).astype(jnp.float32)
    dis = pl.pallas_call(
        _degred_body,
        out_shape=jax.ShapeDtypeStruct((DROWS, 16), jnp.float32),
    )(degp, smat)
    disc = dis.reshape(NROW, 1)
    epk = jnp.stack(
        [src2.reshape(NW, NGRP, GCH, CH),
         dst2.reshape(NW, NGRP, GCH, CH),
         wbits.reshape(NW, NGRP, GCH, CH)], axis=3)
    epk = epk.reshape(NW, NGRP, 3 * GCH, CH)

    znd = jnp.zeros((NROW, D), jnp.float32)

    nblk = N // _ROWS_BLK
    xp = pl.pallas_call(
        _xscale_body,
        grid=(nblk,),
        in_specs=[_row_spec(), _col_spec()],
        out_specs=_row_spec(),
        out_shape=jax.ShapeDtypeStruct((N, D), jnp.float32),
    )(x, disc)

    agg1_parts = _prop(epk, xp, znd)

    h1p = pl.pallas_call(
        _lin_relu_body,
        grid=(nblk,),
        in_specs=[
            _row_spec(), _row_spec(), _col_spec(),
            _full_spec((D, D)), _full_spec((1, D)),
        ],
        out_specs=_row_spec(),
        out_shape=jax.ShapeDtypeStruct((N, D), jnp.float32),
    )(agg1_parts[0], agg1_parts[1], disc, gcn_W1, gcn_b1.reshape(1, D))

    agg2_parts = _prop(epk, h1p, znd)

    wu = [Wu[0:D], Wu[D:2 * D], Wu[2 * D:]]
    wr = [Wr[0:D], Wr[D:2 * D], Wr[2 * D:]]
    wc = [Wc[0:D], Wc[D:2 * D], Wc[2 * D:]]

    out = pl.pallas_call(
        _gru_body,
        grid=(nblk,),
        in_specs=[
            _row_spec(), _row_spec(), _col_spec(), _row_spec(), _row_spec(),
            _full_spec((D, D)), _full_spec((1, D)),
            _full_spec((D, D)), _full_spec((D, D)), _full_spec((D, D)),
            _full_spec((1, D)),
            _full_spec((D, D)), _full_spec((D, D)), _full_spec((D, D)),
            _full_spec((1, D)),
            _full_spec((D, D)), _full_spec((D, D)), _full_spec((D, D)),
            _full_spec((1, D)),
        ],
        out_specs=_row_spec(),
        out_shape=jax.ShapeDtypeStruct((N, D), jnp.float32),
    )(agg2_parts[0], agg2_parts[1], disc, x, h,
      gcn_W2, gcn_b2.reshape(1, D),
      wu[0], wu[1], wu[2], bu.reshape(1, D),
      wr[0], wr[1], wr[2], br.reshape(1, D),
      wc[0], wc[1], wc[2], bc.reshape(1, D))
    return out


# speedup vs baseline: 20.2812x; 1.0295x over previous
import functools

import jax
import jax.numpy as jnp
from jax import lax
from jax.experimental import pallas as pl
from jax.experimental.pallas import tpu as pltpu
from jax.experimental.pallas import tpu_sc as plsc

N = 10000
E = 320000
D = 128

NC = 2
NS = 16
NW = NC * NS
EPW = E // NW
CH = 80
NCHUNK = 128
EPW_PAD = NCHUNK * CH
GCH = 8
NGRP = NCHUNK // GCH
NBUF = 4
NPAD = 10240
DEG_PER_SUB = NPAD // NS
NROW = 10240
AGG_PER_SUB = NROW // NS

_mesh = plsc.VectorSubcoreMesh(core_axis_name="c", subcore_axis_name="s")


DROWS = NPAD * 8 // 128


@functools.partial(
    pl.kernel,
    out_type=jax.ShapeDtypeStruct((NW, DROWS, 128), jnp.float32),
    mesh=_mesh,
    compiler_params=pltpu.CompilerParams(needs_layout_passes=False),
    scratch_types=[
        pltpu.VMEM((80, 128), jnp.int32),
        pltpu.VMEM((80, 128), jnp.float32),
        pltpu.VMEM((DROWS, 128), jnp.float32),
    ],
)
def _deg_kernel(dst_hbm, ew_hbm, out_hbm, dst_v, ew_v, deg8):
    c = lax.axis_index("c")
    s = lax.axis_index("s")
    wid = s * NC + c
    pltpu.sync_copy(dst_hbm.at[wid], dst_v)
    pltpu.sync_copy(ew_hbm.at[wid], ew_v)

    zero16 = jnp.zeros((16,), jnp.float32)

    def z(i, carry):
        for q in range(8):
            deg8[i, pl.ds(q * 16, 16)] = zero16
        return carry

    lax.fori_loop(0, DROWS, z, 0)

    iota = lax.iota(jnp.int32, 16)
    col = iota & 7
    mlow = iota < 8
    mhigh = jnp.logical_not(mlow)

    def acc(i, carry):
        r = i // 8
        q = i % 8
        sl = pl.ds(q * 16, 16)
        dv = dst_v[r, sl]
        wv = ew_v[r, sl]
        addr = dv * 8 + col
        i0 = lax.shift_right_logical(addr, 7)
        i1 = addr & 127
        plsc.addupdate_scatter(deg8, [i0, i1], wv, mask=mlow)
        plsc.addupdate_scatter(deg8, [i0, i1], wv, mask=mhigh)
        return carry

    lax.fori_loop(0, 640, acc, 0, unroll=2)
    pltpu.sync_copy(deg8, out_hbm.at[wid])


def _prop_body(epk, feat_hbm, znd_hbm, agg_out,
               eg0, eg1, r0, r1, r2, r3, x0, x1, x2, x3,
               g0, g1, g2, g3, s0, s1, s2, s3, e0, e1, agg_sh):
    c = lax.axis_index("c")
    s = lax.axis_index("s")
    wid = s * NC + c
    egs = [eg0, eg1]
    rows = [r0, r1, r2, r3]
    sidx = [x0, x1, x2, x3]
    gsem = [g0, g1, g2, g3]
    ssem = [s0, s1, s2, s3]
    esem = [e0, e1]

    sl_sub = pl.ds(s * AGG_PER_SUB, AGG_PER_SUB)
    pltpu.sync_copy(znd_hbm.at[sl_sub], agg_sh.at[sl_sub])

    def e_start(p, g):
        pltpu.async_copy(epk.at[wid, g], egs[p], esem[p])

    def e_wait(p):
        pltpu.make_async_copy(epk.at[wid, 0], egs[p], esem[p]).wait()

    def g_start(b, p, row):
        pltpu.async_copy(feat_hbm.at[egs[p].at[row]], rows[b], gsem[b])

    def g_wait(b):
        pltpu.make_async_copy(feat_hbm.at[eg0.at[0]], rows[b], gsem[b]).wait()

    def s_start(b):
        pltpu.async_copy(rows[b], agg_sh.at[sidx[b]], ssem[b], add=True)

    def s_wait(b):
        pltpu.make_async_copy(rows[b], agg_sh.at[sidx[b]], ssem[b]).wait()

    e_start(0, 0)
    e_wait(0)
    plsc.subcore_barrier()
    g_start(0, 0, 0)
    g_start(1, 0, 3)

    def super_body(t, carry):
        for half in range(2):
            p = half
            for j in range(GCH):
                cc = t * 16 + half * 8 + j
                b = (half * 8 + j) % NBUF
                b2 = (b + 2) % NBUF

                if j == 0:
                    if half == 0:
                        e_start(1, 2 * t + 1)
                    else:
                        @pl.when(t < 7)
                        def _en():
                            e_start(0, 2 * t + 2)
                if j == 6:
                    if half == 0:
                        e_wait(1)
                    else:
                        @pl.when(t < 7)
                        def _ew():
                            e_wait(0)

                pc_j = j + 2
                if pc_j < GCH:
                    prow = 3 * pc_j
                    pparity = p
                else:
                    prow = 3 * (pc_j - GCH)
                    pparity = (p + 1) % 2

                @pl.when(cc + 2 < NCHUNK)
                def _prefetch():
                    @pl.when(cc >= 2)
                    def _drain():
                        s_wait(b2)
                    g_start(b2, pparity, prow)

                g_wait(b)

                drow = 3 * j + 1
                wrow = 3 * j + 2
                for q in range(CH // 16):
                    sl = pl.ds(q * 16, 16)
                    sidx[b][sl] = egs[p][drow, sl]

                def scale(e_, carry2):
                    wsplat = plsc.bitcast(
                        plsc.load_gather(
                            egs[p],
                            [jnp.full((16,), wrow, jnp.int32),
                             jnp.full((16,), e_, jnp.int32)]),
                        jnp.float32)
                    for q in range(8):
                        sl = pl.ds(q * 16, 16)
                        rows[b][e_, sl] = rows[b][e_, sl] * wsplat
                    return carry2

                lax.fori_loop(0, CH, scale, 0, unroll=4)
                s_start(b)
        return carry

    lax.fori_loop(0, NCHUNK // 16, super_body, 0)
    for b in range(NBUF):
        s_wait(b)
    plsc.subcore_barrier()
    pltpu.sync_copy(agg_sh.at[sl_sub], agg_out.at[c, sl_sub])


_prop = functools.partial(
    pl.kernel,
    out_type=jax.ShapeDtypeStruct((NC, NROW, D), jnp.float32),
    mesh=_mesh,
    compiler_params=pltpu.CompilerParams(needs_layout_passes=False),
    scratch_types=[pltpu.VMEM((3 * GCH, CH), jnp.int32)] * 2
    + [pltpu.VMEM((CH, D), jnp.float32)] * NBUF
    + [pltpu.VMEM((CH,), jnp.int32)] * NBUF
    + [pltpu.SemaphoreType.DMA] * (2 * NBUF + 2)
    + [pltpu.VMEM_SHARED((NROW, D), jnp.float32)],
)(_prop_body)


def _degred_body(degp_ref, s_ref, o_ref):
    acc = jnp.sum(degp_ref[...], axis=0)
    deg = jnp.dot(acc, s_ref[...], preferred_element_type=jnp.float32)
    o_ref[...] = lax.rsqrt(jnp.where(deg > 0.0, deg, 1.0))


def _xscale_body(x_ref, disc_ref, o_ref):
    o_ref[...] = x_ref[...] * disc_ref[...]


def _lin_relu_body(p0_ref, p1_ref, disc_ref, w_ref, b_ref, o_ref):
    disc = disc_ref[...]
    agg = (p0_ref[0] + p1_ref[0]) * disc
    y = jnp.dot(agg, w_ref[...], preferred_element_type=jnp.float32)
    o_ref[...] = jnp.maximum(y + b_ref[...], 0.0) * disc


def _gru_body(p0_ref, p1_ref, disc_ref, x_ref, h_ref, w2_ref, b2_ref,
              wu0_ref, wu1_ref, wu2_ref, bu_ref,
              wr0_ref, wr1_ref, wr2_ref, br_ref,
              wc0_ref, wc1_ref, wc2_ref, bc_ref, o_ref):
    agg = (p0_ref[0] + p1_ref[0]) * disc_ref[...]
    x = x_ref[...]
    h = h_ref[...]

    def mm(a, w_ref):
        return jnp.dot(a, w_ref[...], preferred_element_type=jnp.float32)

    g = jax.nn.sigmoid(mm(agg, w2_ref) + b2_ref[...])
    u = jax.nn.sigmoid(mm(x, wu0_ref) + mm(g, wu1_ref) + mm(h, wu2_ref)
                       + bu_ref[...])
    r = jax.nn.sigmoid(mm(x, wr0_ref) + mm(g, wr1_ref) + mm(h, wr2_ref)
                       + br_ref[...])
    cand = jnp.tanh(mm(x, wc0_ref) + mm(g, wc1_ref) + mm(r * h, wc2_ref)
                    + bc_ref[...])
    o_ref[...] = u * h + (1.0 - u) * cand


_ROWS_BLK = 1000


def _row_spec():
    return pl.BlockSpec((_ROWS_BLK, D), lambda i: (i, 0))


def _part_spec(k):
    return pl.BlockSpec((1, _ROWS_BLK, D), lambda i, _k=k: (_k, i, 0))


def _col_spec():
    return pl.BlockSpec((_ROWS_BLK, 1), lambda i: (i, 0))


def _full_spec(shape):
    return pl.BlockSpec(shape, lambda i: tuple(0 for _ in shape))


@jax.jit
def kernel(x, edge_index, edge_weight, h,
           gcn_W1, gcn_b1, gcn_W2, gcn_b2,
           Wu, bu, Wr, br, Wc, bc):
    src = edge_index[0]
    dst = edge_index[1]

    pad_w = EPW_PAD - EPW
    fill_s = jnp.broadcast_to((jnp.arange(pad_w, dtype=jnp.int32) * 37) % N,
                              (NW, pad_w))
    fill_d = jnp.broadcast_to((jnp.arange(pad_w, dtype=jnp.int32) * 53) % NROW,
                              (NW, pad_w))
    src2 = jnp.concatenate([src.reshape(NW, EPW), fill_s], axis=1)
    dst2 = jnp.concatenate([dst.reshape(NW, EPW), fill_d], axis=1)
    w2 = jnp.concatenate(
        [edge_weight.reshape(NW, EPW), jnp.zeros((NW, pad_w), jnp.float32)],
        axis=1)
    wbits = lax.bitcast_convert_type(w2, jnp.int32)

    degp = _deg_kernel(dst2.reshape(NW, 80, 128), w2.reshape(NW, 80, 128))
    smat = (jnp.arange(128, dtype=jnp.int32)[:, None] // 8
            == jnp.arange(16, dtype=jnp.int32)[None, :]).astype(jnp.float32)
    dis = pl.pallas_call(
        _degred_body,
        out_shape=jax.ShapeDtypeStruct((DROWS, 16), jnp.float32),
    )(degp, smat)
    disc = dis.reshape(NROW, 1)
    epk = jnp.stack(
        [src2.reshape(NW, NGRP, GCH, CH),
         dst2.reshape(NW, NGRP, GCH, CH),
         wbits.reshape(NW, NGRP, GCH, CH)], axis=3)
    epk = epk.reshape(NW, NGRP, 3 * GCH, CH)

    znd = jnp.zeros((NROW, D), jnp.float32)

    nblk = N // _ROWS_BLK
    xp = pl.pallas_call(
        _xscale_body,
        grid=(nblk,),
        in_specs=[_row_spec(), _col_spec()],
        out_specs=_row_spec(),
        out_shape=jax.ShapeDtypeStruct((N, D), jnp.float32),
    )(x, disc)

    agg1_parts = _prop(epk, xp, znd)

    h1p = pl.pallas_call(
        _lin_relu_body,
        grid=(nblk,),
        in_specs=[
            _part_spec(0), _part_spec(1), _col_spec(),
            _full_spec((D, D)), _full_spec((1, D)),
        ],
        out_specs=_row_spec(),
        out_shape=jax.ShapeDtypeStruct((N, D), jnp.float32),
    )(agg1_parts, agg1_parts, disc, gcn_W1, gcn_b1.reshape(1, D))

    agg2_parts = _prop(epk, h1p, znd)

    wu = [Wu[0:D], Wu[D:2 * D], Wu[2 * D:]]
    wr = [Wr[0:D], Wr[D:2 * D], Wr[2 * D:]]
    wc = [Wc[0:D], Wc[D:2 * D], Wc[2 * D:]]

    out = pl.pallas_call(
        _gru_body,
        grid=(nblk,),
        in_specs=[
            _part_spec(0), _part_spec(1), _col_spec(), _row_spec(), _row_spec(),
            _full_spec((D, D)), _full_spec((1, D)),
            _full_spec((D, D)), _full_spec((D, D)), _full_spec((D, D)),
            _full_spec((1, D)),
            _full_spec((D, D)), _full_spec((D, D)), _full_spec((D, D)),
            _full_spec((1, D)),
            _full_spec((D, D)), _full_spec((D, D)), _full_spec((D, D)),
            _full_spec((1, D)),
        ],
        out_specs=_row_spec(),
        out_shape=jax.ShapeDtypeStruct((N, D), jnp.float32),
    )(agg2_parts, agg2_parts, disc, x, h,
      gcn_W2, gcn_b2.reshape(1, D),
      wu[0], wu[1], wu[2], bu.reshape(1, D),
      wr[0], wr[1], wr[2], br.reshape(1, D),
      wc[0], wc[1], wc[2], bc.reshape(1, D))
    return out


# speedup vs baseline: 21.5142x; 1.0608x over previous
import functools

import jax
import jax.numpy as jnp
from jax import lax
from jax.experimental import pallas as pl
from jax.experimental.pallas import tpu as pltpu
from jax.experimental.pallas import tpu_sc as plsc

N = 10000
E = 320000
D = 128

NC = 2
NS = 16
NW = NC * NS
EPW = E // NW
CH = 80
NCHUNK = 128
EPW_PAD = NCHUNK * CH
GCH = 8
NGRP = NCHUNK // GCH
NBUF = 4
NPAD = 10240
DEG_PER_SUB = NPAD // NS
NROW = 10240
AGG_PER_SUB = NROW // NS

_mesh = plsc.VectorSubcoreMesh(core_axis_name="c", subcore_axis_name="s")


DROWS = NPAD * 8 // 128


@functools.partial(
    pl.kernel,
    out_type=jax.ShapeDtypeStruct((NW, DROWS, 128), jnp.float32),
    mesh=_mesh,
    compiler_params=pltpu.CompilerParams(needs_layout_passes=False),
    scratch_types=[
        pltpu.VMEM((80, 128), jnp.int32),
        pltpu.VMEM((80, 128), jnp.float32),
        pltpu.VMEM((DROWS, 128), jnp.float32),
    ],
)
def _deg_kernel(dst_hbm, ew_hbm, out_hbm, dst_v, ew_v, deg8):
    c = lax.axis_index("c")
    s = lax.axis_index("s")
    wid = s * NC + c
    pltpu.sync_copy(dst_hbm.at[wid], dst_v)
    pltpu.sync_copy(ew_hbm.at[wid], ew_v)

    zero16 = jnp.zeros((16,), jnp.float32)

    def z(i, carry):
        for q in range(8):
            deg8[i, pl.ds(q * 16, 16)] = zero16
        return carry

    lax.fori_loop(0, DROWS, z, 0)

    iota = lax.iota(jnp.int32, 16)
    col = iota & 7
    mlow = iota < 8
    mhigh = jnp.logical_not(mlow)

    def acc(i, carry):
        r = i // 8
        q = i % 8
        sl = pl.ds(q * 16, 16)
        dv = dst_v[r, sl]
        wv = ew_v[r, sl]
        addr = dv * 8 + col
        i0 = lax.shift_right_logical(addr, 7)
        i1 = addr & 127
        plsc.addupdate_scatter(deg8, [i0, i1], wv, mask=mlow)
        plsc.addupdate_scatter(deg8, [i0, i1], wv, mask=mhigh)
        return carry

    lax.fori_loop(0, 640, acc, 0, unroll=2)
    pltpu.sync_copy(deg8, out_hbm.at[wid])


def _prop_body(spk, dpk, wpk, feat_hbm, znd_hbm, agg_out,
               sg0, sg1, dg0, dg1, wg0, wg1,
               r0, r1, r2, r3, x0, x1, x2, x3,
               g0, g1, g2, g3, s0, s1, s2, s3, e0, e1, agg_sh):
    c = lax.axis_index("c")
    s = lax.axis_index("s")
    wid = s * NC + c
    sgs = [sg0, sg1]
    dgs = [dg0, dg1]
    wgs = [wg0, wg1]
    rows = [r0, r1, r2, r3]
    sidx = [x0, x1, x2, x3]
    gsem = [g0, g1, g2, g3]
    ssem = [s0, s1, s2, s3]
    esem = [e0, e1]

    sl_sub = pl.ds(s * AGG_PER_SUB, AGG_PER_SUB)
    pltpu.sync_copy(znd_hbm.at[sl_sub], agg_sh.at[sl_sub])

    def e_start(p, g):
        pltpu.async_copy(spk.at[wid, g], sgs[p], esem[p])
        pltpu.async_copy(dpk.at[wid, g], dgs[p], esem[p])
        pltpu.async_copy(wpk.at[wid, g], wgs[p], esem[p])

    def e_wait(p):
        pltpu.make_async_copy(spk.at[wid, 0], sgs[p], esem[p]).wait()
        pltpu.make_async_copy(dpk.at[wid, 0], dgs[p], esem[p]).wait()
        pltpu.make_async_copy(wpk.at[wid, 0], wgs[p], esem[p]).wait()

    def g_start(b, p, row):
        pltpu.async_copy(feat_hbm.at[sgs[p].at[row]], rows[b], gsem[b])

    def g_wait(b):
        pltpu.make_async_copy(feat_hbm.at[sg0.at[0]], rows[b], gsem[b]).wait()

    def s_start(b):
        pltpu.async_copy(rows[b], agg_sh.at[sidx[b]], ssem[b], add=True)

    def s_wait(b):
        pltpu.make_async_copy(rows[b], agg_sh.at[sidx[b]], ssem[b]).wait()

    e_start(0, 0)
    e_wait(0)
    plsc.subcore_barrier()
    g_start(0, 0, 0)
    g_start(1, 0, 1)

    def super_body(t, carry):
        for half in range(2):
            p = half
            for j in range(GCH):
                cc = t * 16 + half * 8 + j
                b = (half * 8 + j) % NBUF
                b2 = (b + 2) % NBUF

                if j == 0:
                    if half == 0:
                        e_start(1, 2 * t + 1)
                    else:
                        @pl.when(t < 7)
                        def _en():
                            e_start(0, 2 * t + 2)
                if j == 6:
                    if half == 0:
                        e_wait(1)
                    else:
                        @pl.when(t < 7)
                        def _ew():
                            e_wait(0)

                pc_j = j + 2
                if pc_j < GCH:
                    prow = pc_j
                    pparity = p
                else:
                    prow = pc_j - GCH
                    pparity = (p + 1) % 2

                @pl.when(cc + 2 < NCHUNK)
                def _prefetch():
                    @pl.when(cc >= 2)
                    def _drain():
                        s_wait(b2)
                    g_start(b2, pparity, prow)

                g_wait(b)

                for q in range(CH // 16):
                    sl = pl.ds(q * 16, 16)
                    sidx[b][sl] = dgs[p][j, sl]

                def scale(e_, carry2):
                    wsplat = plsc.bitcast(
                        plsc.load_gather(
                            wgs[p],
                            [jnp.full((16,), j, jnp.int32),
                             jnp.full((16,), e_, jnp.int32)]),
                        jnp.float32)
                    for q in range(8):
                        sl = pl.ds(q * 16, 16)
                        rows[b][e_, sl] = rows[b][e_, sl] * wsplat
                    return carry2

                lax.fori_loop(0, CH, scale, 0, unroll=4)
                s_start(b)
        return carry

    lax.fori_loop(0, NCHUNK // 16, super_body, 0)
    for b in range(NBUF):
        s_wait(b)
    plsc.subcore_barrier()
    pltpu.sync_copy(agg_sh.at[sl_sub], agg_out.at[c, sl_sub])


_prop = functools.partial(
    pl.kernel,
    out_type=jax.ShapeDtypeStruct((NC, NROW, D), jnp.float32),
    mesh=_mesh,
    compiler_params=pltpu.CompilerParams(needs_layout_passes=False),
    scratch_types=[pltpu.VMEM((GCH, CH), jnp.int32)] * 6
    + [pltpu.VMEM((CH, D), jnp.float32)] * NBUF
    + [pltpu.VMEM((CH,), jnp.int32)] * NBUF
    + [pltpu.SemaphoreType.DMA] * (2 * NBUF + 2)
    + [pltpu.VMEM_SHARED((NROW, D), jnp.float32)],
)(_prop_body)


def _degx_body(degp_ref, s_ref, x_ref, dis_ref, xp_ref):
    acc = jnp.sum(degp_ref[...], axis=0)
    deg = jnp.dot(acc, s_ref[...], preferred_element_type=jnp.float32)
    dis = lax.rsqrt(jnp.where(deg > 0.0, deg, 1.0))
    dis_ref[...] = dis
    x3 = x_ref[...].reshape(N // 16, 16, D)
    xp_ref[...] = (x3 * dis[:N // 16, :, None]).reshape(N, D)


def _lin_relu_body(p0_ref, p1_ref, disc_ref, w_ref, b_ref, o_ref):
    disc = disc_ref[...]
    agg = (p0_ref[0] + p1_ref[0]) * disc
    y = jnp.dot(agg, w_ref[...], preferred_element_type=jnp.float32)
    o_ref[...] = jnp.maximum(y + b_ref[...], 0.0) * disc


def _gru_body(p0_ref, p1_ref, disc_ref, x_ref, h_ref, w2_ref, b2_ref,
              wu0_ref, wu1_ref, wu2_ref, bu_ref,
              wr0_ref, wr1_ref, wr2_ref, br_ref,
              wc0_ref, wc1_ref, wc2_ref, bc_ref, o_ref):
    agg = (p0_ref[0] + p1_ref[0]) * disc_ref[...]
    x = x_ref[...]
    h = h_ref[...]

    def mm(a, w_ref):
        return jnp.dot(a, w_ref[...], preferred_element_type=jnp.float32)

    g = jax.nn.sigmoid(mm(agg, w2_ref) + b2_ref[...])
    u = jax.nn.sigmoid(mm(x, wu0_ref) + mm(g, wu1_ref) + mm(h, wu2_ref)
                       + bu_ref[...])
    r = jax.nn.sigmoid(mm(x, wr0_ref) + mm(g, wr1_ref) + mm(h, wr2_ref)
                       + br_ref[...])
    cand = jnp.tanh(mm(x, wc0_ref) + mm(g, wc1_ref) + mm(r * h, wc2_ref)
                    + bc_ref[...])
    o_ref[...] = u * h + (1.0 - u) * cand


_ROWS_BLK = 1000


def _row_spec():
    return pl.BlockSpec((_ROWS_BLK, D), lambda i: (i, 0))


def _part_spec(k):
    return pl.BlockSpec((1, _ROWS_BLK, D), lambda i, _k=k: (_k, i, 0))


def _col_spec():
    return pl.BlockSpec((_ROWS_BLK, 1), lambda i: (i, 0))


def _full_spec(shape):
    return pl.BlockSpec(shape, lambda i: tuple(0 for _ in shape))


@jax.jit
def kernel(x, edge_index, edge_weight, h,
           gcn_W1, gcn_b1, gcn_W2, gcn_b2,
           Wu, bu, Wr, br, Wc, bc):
    src = edge_index[0]
    dst = edge_index[1]

    pad_w = EPW_PAD - EPW
    fill_s = jnp.broadcast_to((jnp.arange(pad_w, dtype=jnp.int32) * 37) % N,
                              (NW, pad_w))
    fill_d = jnp.broadcast_to((jnp.arange(pad_w, dtype=jnp.int32) * 53) % NROW,
                              (NW, pad_w))
    src2 = jnp.concatenate([src.reshape(NW, EPW), fill_s], axis=1)
    dst2 = jnp.concatenate([dst.reshape(NW, EPW), fill_d], axis=1)
    w2 = jnp.concatenate(
        [edge_weight.reshape(NW, EPW), jnp.zeros((NW, pad_w), jnp.float32)],
        axis=1)
    wbits = lax.bitcast_convert_type(w2, jnp.int32)

    degp = _deg_kernel(dst2.reshape(NW, 80, 128), w2.reshape(NW, 80, 128))
    smat = (jnp.arange(128, dtype=jnp.int32)[:, None] // 8
            == jnp.arange(16, dtype=jnp.int32)[None, :]).astype(jnp.float32)
    dis, xp = pl.pallas_call(
        _degx_body,
        out_shape=[
            jax.ShapeDtypeStruct((DROWS, 16), jnp.float32),
            jax.ShapeDtypeStruct((N, D), jnp.float32),
        ],
    )(degp, smat, x)
    disc = dis.reshape(NROW, 1)
    spk = src2.reshape(NW, NGRP, GCH, CH)
    dpk = dst2.reshape(NW, NGRP, GCH, CH)
    wpk = wbits.reshape(NW, NGRP, GCH, CH)

    znd = jnp.zeros((NROW, D), jnp.float32)
    nblk = N // _ROWS_BLK

    agg1_parts = _prop(spk, dpk, wpk, xp, znd)

    h1p = pl.pallas_call(
        _lin_relu_body,
        grid=(nblk,),
        in_specs=[
            _part_spec(0), _part_spec(1), _col_spec(),
            _full_spec((D, D)), _full_spec((1, D)),
        ],
        out_specs=_row_spec(),
        out_shape=jax.ShapeDtypeStruct((N, D), jnp.float32),
    )(agg1_parts, agg1_parts, disc, gcn_W1, gcn_b1.reshape(1, D))

    agg2_parts = _prop(spk, dpk, wpk, h1p, znd)

    wu = [Wu[0:D], Wu[D:2 * D], Wu[2 * D:]]
    wr = [Wr[0:D], Wr[D:2 * D], Wr[2 * D:]]
    wc = [Wc[0:D], Wc[D:2 * D], Wc[2 * D:]]

    out = pl.pallas_call(
        _gru_body,
        grid=(nblk,),
        in_specs=[
            _part_spec(0), _part_spec(1), _col_spec(), _row_spec(), _row_spec(),
            _full_spec((D, D)), _full_spec((1, D)),
            _full_spec((D, D)), _full_spec((D, D)), _full_spec((D, D)),
            _full_spec((1, D)),
            _full_spec((D, D)), _full_spec((D, D)), _full_spec((D, D)),
            _full_spec((1, D)),
            _full_spec((D, D)), _full_spec((D, D)), _full_spec((D, D)),
            _full_spec((1, D)),
        ],
        out_specs=_row_spec(),
        out_shape=jax.ShapeDtypeStruct((N, D), jnp.float32),
    )(agg2_parts, agg2_parts, disc, x, h,
      gcn_W2, gcn_b2.reshape(1, D),
      wu[0], wu[1], wu[2], bu.reshape(1, D),
      wr[0], wr[1], wr[2], br.reshape(1, D),
      wc[0], wc[1], wc[2], bc.reshape(1, D))
    return out


# speedup vs baseline: 21.8927x; 1.0176x over previous
import functools

import jax
import jax.numpy as jnp
from jax import lax
from jax.experimental import pallas as pl
from jax.experimental.pallas import tpu as pltpu
from jax.experimental.pallas import tpu_sc as plsc

N = 10000
E = 320000
D = 128

NC = 2
NS = 16
NW = NC * NS
EPW = E // NW
CH = 80
NCHUNK = 128
EPW_PAD = NCHUNK * CH
GCH = 8
NGRP = NCHUNK // GCH
NBUF = 4
NPAD = 10240
DEG_PER_SUB = NPAD // NS
NROW = 10240
AGG_PER_SUB = NROW // NS

_mesh = plsc.VectorSubcoreMesh(core_axis_name="c", subcore_axis_name="s")


DROWS = NPAD * 8 // 128


@functools.partial(
    pl.kernel,
    out_type=jax.ShapeDtypeStruct((NW, DROWS, 128), jnp.float32),
    mesh=_mesh,
    compiler_params=pltpu.CompilerParams(needs_layout_passes=False),
    scratch_types=[
        pltpu.VMEM((80, 128), jnp.int32),
        pltpu.VMEM((80, 128), jnp.float32),
        pltpu.VMEM((DROWS, 128), jnp.float32),
    ],
)
def _deg_kernel(dst_hbm, ew_hbm, out_hbm, dst_v, ew_v, deg8):
    c = lax.axis_index("c")
    s = lax.axis_index("s")
    wid = s * NC + c
    pltpu.sync_copy(dst_hbm.at[wid], dst_v)
    pltpu.sync_copy(ew_hbm.at[wid], ew_v)

    zero16 = jnp.zeros((16,), jnp.float32)

    def z(i, carry):
        for q in range(8):
            deg8[i, pl.ds(q * 16, 16)] = zero16
        return carry

    lax.fori_loop(0, DROWS, z, 0)

    iota = lax.iota(jnp.int32, 16)
    col = iota & 7
    mlow = iota < 8
    mhigh = jnp.logical_not(mlow)

    def acc(i, carry):
        r = i // 8
        q = i % 8
        sl = pl.ds(q * 16, 16)
        dv = dst_v[r, sl]
        wv = ew_v[r, sl]
        addr = dv * 8 + col
        i0 = lax.shift_right_logical(addr, 7)
        i1 = addr & 127
        plsc.addupdate_scatter(deg8, [i0, i1], wv, mask=mlow)
        plsc.addupdate_scatter(deg8, [i0, i1], wv, mask=mhigh)
        return carry

    lax.fori_loop(0, 640, acc, 0, unroll=2)
    pltpu.sync_copy(deg8, out_hbm.at[wid])


def _prop_body(spk, dpk, wpk, feat_hbm, znd_hbm, agg_out,
               sg0, sg1, dg0, dg1, wg0, wg1,
               r0, r1, r2, r3,
               g0, g1, g2, g3, s0, s1, s2, s3, e0, e1, agg_sh):
    c = lax.axis_index("c")
    s = lax.axis_index("s")
    wid = s * NC + c
    sgs = [sg0, sg1]
    dgs = [dg0, dg1]
    wgs = [wg0, wg1]
    rows = [r0, r1, r2, r3]
    gsem = [g0, g1, g2, g3]
    ssem = [s0, s1, s2, s3]
    esem = [e0, e1]

    sl_sub = pl.ds(s * AGG_PER_SUB, AGG_PER_SUB)

    def e_start(p, g):
        pltpu.async_copy(spk.at[wid, g], sgs[p], esem[p])
        pltpu.async_copy(dpk.at[wid, g], dgs[p], esem[p])
        pltpu.async_copy(wpk.at[wid, g], wgs[p], esem[p])

    def e_wait(p):
        pltpu.make_async_copy(spk.at[wid, 0], sgs[p], esem[p]).wait()
        pltpu.make_async_copy(dpk.at[wid, 0], dgs[p], esem[p]).wait()
        pltpu.make_async_copy(wpk.at[wid, 0], wgs[p], esem[p]).wait()

    def g_start(b, p, row):
        pltpu.async_copy(feat_hbm.at[sgs[p].at[row]], rows[b], gsem[b])

    def g_wait(b):
        pltpu.make_async_copy(feat_hbm.at[sg0.at[0]], rows[b], gsem[b]).wait()

    def s_start(b, p, row):
        pltpu.async_copy(rows[b], agg_sh.at[dgs[p].at[row]], ssem[b], add=True)

    def s_wait(b):
        pltpu.make_async_copy(rows[b], agg_sh.at[dg0.at[0]], ssem[b]).wait()

    zdesc = pltpu.make_async_copy(znd_hbm.at[sl_sub], agg_sh.at[sl_sub],
                                  gsem[2])
    zdesc.start()
    e_start(0, 0)
    e_wait(0)
    g_start(0, 0, 0)
    g_start(1, 0, 1)
    zdesc.wait()
    plsc.subcore_barrier()

    def super_body(t, carry):
        for half in range(2):
            p = half
            for j in range(GCH):
                cc = t * 16 + half * 8 + j
                b = (half * 8 + j) % NBUF
                b2 = (b + 2) % NBUF

                if j == 2:
                    if half == 0:
                        e_start(1, 2 * t + 1)
                    else:
                        @pl.when(t < 7)
                        def _en():
                            e_start(0, 2 * t + 2)
                if j == 6:
                    if half == 0:
                        e_wait(1)
                    else:
                        @pl.when(t < 7)
                        def _ew():
                            e_wait(0)

                pc_j = j + 2
                if pc_j < GCH:
                    prow = pc_j
                    pparity = p
                else:
                    prow = pc_j - GCH
                    pparity = (p + 1) % 2

                @pl.when(cc + 2 < NCHUNK)
                def _prefetch():
                    @pl.when(cc >= 2)
                    def _drain():
                        s_wait(b2)
                    g_start(b2, pparity, prow)

                g_wait(b)

                def scale(e_, carry2):
                    wsplat = plsc.bitcast(
                        plsc.load_gather(
                            wgs[p],
                            [jnp.full((16,), j, jnp.int32),
                             jnp.full((16,), e_, jnp.int32)]),
                        jnp.float32)
                    for q in range(8):
                        sl = pl.ds(q * 16, 16)
                        rows[b][e_, sl] = rows[b][e_, sl] * wsplat
                    return carry2

                lax.fori_loop(0, CH, scale, 0, unroll=4)
                s_start(b, p, j)
        return carry

    lax.fori_loop(0, NCHUNK // 16, super_body, 0)
    for b in range(NBUF):
        s_wait(b)
    plsc.subcore_barrier()
    pltpu.sync_copy(agg_sh.at[sl_sub], agg_out.at[c, sl_sub])


_prop = functools.partial(
    pl.kernel,
    out_type=jax.ShapeDtypeStruct((NC, NROW, D), jnp.float32),
    mesh=_mesh,
    compiler_params=pltpu.CompilerParams(needs_layout_passes=False),
    scratch_types=[pltpu.VMEM((GCH, CH), jnp.int32)] * 6
    + [pltpu.VMEM((CH, D), jnp.float32)] * NBUF
    + [pltpu.SemaphoreType.DMA] * (2 * NBUF + 2)
    + [pltpu.VMEM_SHARED((NROW, D), jnp.float32)],
)(_prop_body)


def _degx_body(degp_ref, s_ref, x_ref, dis_ref, xp_ref):
    acc = jnp.sum(degp_ref[...], axis=0)
    deg = jnp.dot(acc, s_ref[...], preferred_element_type=jnp.float32)
    dis = lax.rsqrt(jnp.where(deg > 0.0, deg, 1.0))
    dis_ref[...] = dis
    x3 = x_ref[...].reshape(N // 16, 16, D)
    xp_ref[...] = (x3 * dis[:N // 16, :, None]).reshape(N, D)


def _lin_relu_body(p0_ref, p1_ref, disc_ref, w_ref, b_ref, o_ref):
    disc = disc_ref[...]
    agg = (p0_ref[0] + p1_ref[0]) * disc
    y = jnp.dot(agg, w_ref[...], preferred_element_type=jnp.float32)
    o_ref[...] = jnp.maximum(y + b_ref[...], 0.0) * disc


def _gru_body(p0_ref, p1_ref, disc_ref, x_ref, h_ref, w2_ref, b2_ref,
              wu0_ref, wu1_ref, wu2_ref, bu_ref,
              wr0_ref, wr1_ref, wr2_ref, br_ref,
              wc0_ref, wc1_ref, wc2_ref, bc_ref, o_ref):
    agg = (p0_ref[0] + p1_ref[0]) * disc_ref[...]
    x = x_ref[...]
    h = h_ref[...]

    def mm(a, w_ref):
        return jnp.dot(a, w_ref[...], preferred_element_type=jnp.float32)

    g = jax.nn.sigmoid(mm(agg, w2_ref) + b2_ref[...])
    u = jax.nn.sigmoid(mm(x, wu0_ref) + mm(g, wu1_ref) + mm(h, wu2_ref)
                       + bu_ref[...])
    r = jax.nn.sigmoid(mm(x, wr0_ref) + mm(g, wr1_ref) + mm(h, wr2_ref)
                       + br_ref[...])
    cand = jnp.tanh(mm(x, wc0_ref) + mm(g, wc1_ref) + mm(r * h, wc2_ref)
                    + bc_ref[...])
    o_ref[...] = u * h + (1.0 - u) * cand


_ROWS_BLK = 1000


def _row_spec():
    return pl.BlockSpec((_ROWS_BLK, D), lambda i: (i, 0))


def _part_spec(k):
    return pl.BlockSpec((1, _ROWS_BLK, D), lambda i, _k=k: (_k, i, 0))


def _col_spec():
    return pl.BlockSpec((_ROWS_BLK, 1), lambda i: (i, 0))


def _full_spec(shape):
    return pl.BlockSpec(shape, lambda i: tuple(0 for _ in shape))


@jax.jit
def kernel(x, edge_index, edge_weight, h,
           gcn_W1, gcn_b1, gcn_W2, gcn_b2,
           Wu, bu, Wr, br, Wc, bc):
    src = edge_index[0]
    dst = edge_index[1]

    pad_w = EPW_PAD - EPW
    fill_s = jnp.broadcast_to((jnp.arange(pad_w, dtype=jnp.int32) * 37) % N,
                              (NW, pad_w))
    fill_d = jnp.broadcast_to((jnp.arange(pad_w, dtype=jnp.int32) * 53) % NROW,
                              (NW, pad_w))
    src2 = jnp.concatenate([src.reshape(NW, EPW), fill_s], axis=1)
    dst2 = jnp.concatenate([dst.reshape(NW, EPW), fill_d], axis=1)
    w2 = jnp.concatenate(
        [edge_weight.reshape(NW, EPW), jnp.zeros((NW, pad_w), jnp.float32)],
        axis=1)
    wbits = lax.bitcast_convert_type(w2, jnp.int32)

    degp = _deg_kernel(dst2.reshape(NW, 80, 128), w2.reshape(NW, 80, 128))
    smat = (jnp.arange(128, dtype=jnp.int32)[:, None] // 8
            == jnp.arange(16, dtype=jnp.int32)[None, :]).astype(jnp.float32)
    dis, xp = pl.pallas_call(
        _degx_body,
        out_shape=[
            jax.ShapeDtypeStruct((DROWS, 16), jnp.float32),
            jax.ShapeDtypeStruct((N, D), jnp.float32),
        ],
    )(degp, smat, x)
    disc = dis.reshape(NROW, 1)
    spk = src2.reshape(NW, NGRP, GCH, CH)
    dpk = dst2.reshape(NW, NGRP, GCH, CH)
    wpk = wbits.reshape(NW, NGRP, GCH, CH)

    znd = jnp.zeros((NROW, D), jnp.float32)
    nblk = N // _ROWS_BLK

    agg1_parts = _prop(spk, dpk, wpk, xp, znd)

    h1p = pl.pallas_call(
        _lin_relu_body,
        grid=(nblk,),
        in_specs=[
            _part_spec(0), _part_spec(1), _col_spec(),
            _full_spec((D, D)), _full_spec((1, D)),
        ],
        out_specs=_row_spec(),
        out_shape=jax.ShapeDtypeStruct((N, D), jnp.float32),
    )(agg1_parts, agg1_parts, disc, gcn_W1, gcn_b1.reshape(1, D))

    agg2_parts = _prop(spk, dpk, wpk, h1p, znd)

    wu = [Wu[0:D], Wu[D:2 * D], Wu[2 * D:]]
    wr = [Wr[0:D], Wr[D:2 * D], Wr[2 * D:]]
    wc = [Wc[0:D], Wc[D:2 * D], Wc[2 * D:]]

    out = pl.pallas_call(
        _gru_body,
        grid=(nblk,),
        in_specs=[
            _part_spec(0), _part_spec(1), _col_spec(), _row_spec(), _row_spec(),
            _full_spec((D, D)), _full_spec((1, D)),
            _full_spec((D, D)), _full_spec((D, D)), _full_spec((D, D)),
            _full_spec((1, D)),
            _full_spec((D, D)), _full_spec((D, D)), _full_spec((D, D)),
            _full_spec((1, D)),
            _full_spec((D, D)), _full_spec((D, D)), _full_spec((D, D)),
            _full_spec((1, D)),
        ],
        out_specs=_row_spec(),
        out_shape=jax.ShapeDtypeStruct((N, D), jnp.float32),
    )(agg2_parts, agg2_parts, disc, x, h,
      gcn_W2, gcn_b2.reshape(1, D),
      wu[0], wu[1], wu[2], bu.reshape(1, D),
      wr[0], wr[1], wr[2], br.reshape(1, D),
      wc[0], wc[1], wc[2], bc.reshape(1, D))
    return out
